# baseline (speedup 1.0000x reference)
import dataclasses
import functools

import jax
import jax.numpy as jnp
from jax import lax
from jax.experimental import pallas as pl
from jax.experimental.pallas import tpu as pltpu
from jax.experimental.pallas import tpu_sc as plsc

E = 8
TOPK = 2
D = 1024
DI = 512
EPS = 1e-06

T = 2048
RT = 256
M = 128
NT = 4992
TILES = NT // M
NSUB = 16
NCORE = 2


def _sc_params(tc_tiling=False):
    cp = pltpu.CompilerParams()
    if "needs_layout_passes" in pltpu.CompilerParams.__dataclass_fields__:
        cp = dataclasses.replace(cp, needs_layout_passes=False)
    if tc_tiling:
        cp = dataclasses.replace(cp, use_tc_tiling_on_sc=True)
    return cp


def _tdot(a, b):
    return jax.lax.dot_general(
        a, b, (((1,), (1,)), ((), ())), preferred_element_type=jnp.float32)


def _router_body(x_ref, rmsw_ref, rw_ref,
                 xn_ref, logits_ref, pe_ref, w1_ref, w2_ref):
    x = x_ref[...]
    var = jnp.mean(x * x, axis=1, keepdims=True)
    xn = x * jax.lax.rsqrt(var + EPS) * rmsw_ref[...]
    xn_ref[...] = xn.astype(jnp.bfloat16)

    logits = jax.lax.dot_general(
        xn, rw_ref[...], (((1,), (1,)), ((), ())),
        preferred_element_type=jnp.float32)
    logits_ref[...] = logits

    m = jnp.max(logits, axis=1, keepdims=True)
    ex = jnp.exp(logits - m)
    w = ex / jnp.sum(ex, axis=1, keepdims=True)
    iota = jax.lax.broadcasted_iota(jnp.int32, (RT, E), 1)
    m1 = jnp.max(w, axis=1, keepdims=True)
    i1 = jnp.min(jnp.where(w == m1, iota, E), axis=1, keepdims=True)
    w2_ = jnp.where(iota == i1, -jnp.inf, w)
    m2 = jnp.max(w2_, axis=1, keepdims=True)
    i2 = jnp.min(jnp.where(w2_ == m2, iota, E), axis=1, keepdims=True)

    pe_ref[...] = jnp.reshape(i1 + 8 * i2, (1, 1, RT))
    w1_ref[...] = jnp.reshape(m1, (1, 1, RT))
    w2_ref[...] = jnp.reshape(m2, (1, 1, RT))


def _router(x, rmsw, rw):
    return pl.pallas_call(
        _router_body,
        grid=(T // RT,),
        in_specs=[
            pl.BlockSpec((RT, D), lambda i: (i, 0)),
            pl.BlockSpec((1, D), lambda i: (0, 0)),
            pl.BlockSpec((E, D), lambda i: (0, 0)),
        ],
        out_specs=[
            pl.BlockSpec((RT, D), lambda i: (i, 0)),
            pl.BlockSpec((RT, E), lambda i: (i, 0)),
            pl.BlockSpec((1, 1, RT), lambda i: (i, 0, 0)),
            pl.BlockSpec((1, 1, RT), lambda i: (i, 0, 0)),
            pl.BlockSpec((1, 1, RT), lambda i: (i, 0, 0)),
        ],
        out_shape=[
            jax.ShapeDtypeStruct((T, D), jnp.bfloat16),
            jax.ShapeDtypeStruct((T, E), jnp.float32),
            jax.ShapeDtypeStruct((T // RT, 1, RT), jnp.int32),
            jax.ShapeDtypeStruct((T // RT, 1, RT), jnp.float32),
            jax.ShapeDtypeStruct((T // RT, 1, RT), jnp.float32),
        ],
    )(x, rmsw, rw)


def _dispatch_body(pe_hbm, xn_hbm, xs_hbm, meta_hbm, pos_hbm,
                   pe_v, ev0_v, ev1_v, dest0_v, dest1_v, h_v, histl_v,
                   meta_v, rows_v, hist_sh):
    c = lax.axis_index("c")
    s = lax.axis_index("s")
    lane = jax.lax.broadcasted_iota(jnp.int32, (NSUB,), 0)

    pltpu.sync_copy(pe_hbm.at[pl.ds(s * 128, 128)], pe_v)

    h = jnp.zeros((NSUB,), jnp.int32)
    for v in range(8):
        pv = pe_v[pl.ds(16 * v, 16)]
        e0 = jnp.bitwise_and(pv, 7)
        e1 = jnp.right_shift(pv, 3)
        ev0_v[pl.ds(16 * v, 16)] = e0
        ev1_v[pl.ds(16 * v, 16)] = e1
        for b in range(E):
            h = h + jnp.where(lane == b,
                              plsc.all_reduce_population_count(e0 == b)
                              + plsc.all_reduce_population_count(e1 == b), 0)
    h_v[...] = h
    pltpu.sync_copy(h_v, hist_sh.at[pl.ds(s * NSUB, NSUB)])
    plsc.subcore_barrier()
    pltpu.sync_copy(hist_sh, histl_v)

    ghist = jnp.zeros((NSUB,), jnp.int32)
    before = jnp.zeros((NSUB,), jnp.int32)
    for w in range(NSUB):
        row = histl_v[pl.ds(w * NSUB, NSUB)]
        ghist = ghist + row
        before = before + jnp.where(w < s, row, 0)
    pc = jnp.bitwise_and(ghist + (M - 1), -M)
    csum = plsc.cumsum(pc)
    base = (csum - pc) + before

    for v in range(8):
        dest0_v[pl.ds(16 * v, 16)] = jnp.zeros((16,), jnp.int32)
        dest1_v[pl.ds(16 * v, 16)] = jnp.zeros((16,), jnp.int32)
    for b in range(E):
        run = base[b]
        for v in range(16):
            dv = dest0_v if v < 8 else dest1_v
            evv = ev0_v if v < 8 else ev1_v
            sl = pl.ds(16 * (v % 8), 16)
            ev = evv[sl]
            mk = ev == b
            cs = plsc.cumsum(mk.astype(jnp.int32))
            dv[sl] = jnp.where(mk, run + cs - 1, dv[sl])
            run = run + jnp.max(cs)

    @pl.when((c == 0) & (s == 0))
    def _meta():
        for g in range(3):
            tstart = (lane + 16 * g) * M
            acc = jnp.zeros((NSUB,), jnp.int32)
            for b in range(E):
                acc = acc + (tstart >= csum[b]).astype(jnp.int32)
            meta_v[pl.ds(16 * g, 16)] = jnp.minimum(acc, E - 1)
        pltpu.sync_copy(meta_v, meta_hbm)

    pltpu.sync_copy(xn_hbm.at[pl.ds(s * 128, 128)], rows_v)

    @pl.when(c == 0)
    def _scatter0():
        pltpu.sync_copy(rows_v, xs_hbm.at[dest0_v])
        pltpu.sync_copy(dest0_v, pos_hbm.at[pl.ds(s * 128, 128)])

    @pl.when(c == 1)
    def _scatter1():
        pltpu.sync_copy(rows_v, xs_hbm.at[dest1_v])
        pltpu.sync_copy(dest1_v, pos_hbm.at[pl.ds(T + s * 128, 128)])


def _dispatch(pe, xn3):
    mesh = plsc.VectorSubcoreMesh(core_axis_name="c", subcore_axis_name="s")
    f = pl.kernel(
        _dispatch_body,
        out_type=[
            jax.ShapeDtypeStruct((NT, D // 2), jnp.int32),
            jax.ShapeDtypeStruct((48,), jnp.int32),
            jax.ShapeDtypeStruct((2 * T,), jnp.int32),
        ],
        mesh=mesh,
        scratch_types=[
            pltpu.VMEM((128,), jnp.int32),
            pltpu.VMEM((128,), jnp.int32),
            pltpu.VMEM((128,), jnp.int32),
            pltpu.VMEM((128,), jnp.int32),
            pltpu.VMEM((128,), jnp.int32),
            pltpu.VMEM((NSUB,), jnp.int32),
            pltpu.VMEM((NSUB * NSUB,), jnp.int32),
            pltpu.VMEM((48,), jnp.int32),
            pltpu.VMEM((128, D // 2), jnp.int32),
            pltpu.VMEM_SHARED((NSUB * NSUB,), jnp.int32),
        ],
        compiler_params=_sc_params(tc_tiling=True),
    )
    return f(pe, xn3)


def _gmm_body(meta_ref, xs_ref, wg_ref, wu_ref, wd_ref, y_ref,
              wgb_ref, wub_ref, wdb_ref):
    j = pl.program_id(0)
    jm = jnp.maximum(j - 1, 0)
    changed = jnp.logical_or(j == 0, meta_ref[j] != meta_ref[jm])

    @pl.when(changed)
    def _cast_gu():
        wgb_ref[...] = wg_ref[0].astype(jnp.bfloat16)
        wub_ref[...] = wu_ref[0].astype(jnp.bfloat16)

    xb = xs_ref[...]
    g = _tdot(xb, wgb_ref[...])
    u = _tdot(xb, wub_ref[...])

    @pl.when(changed)
    def _cast_d():
        wdb_ref[...] = wd_ref[0].astype(jnp.bfloat16)

    h = ((g * jax.nn.sigmoid(g)) * u).astype(jnp.bfloat16)
    y_ref[...] = _tdot(h, wdb_ref[...])


def _gmm(meta, xs2, w_gate, w_up, w_down):
    return pl.pallas_call(
        _gmm_body,
        grid_spec=pltpu.PrefetchScalarGridSpec(
            num_scalar_prefetch=1,
            grid=(TILES,),
            in_specs=[
                pl.BlockSpec((M, D), lambda j, m: (j, 0)),
                pl.BlockSpec((1, DI, D), lambda j, m: (m[j], 0, 0)),
                pl.BlockSpec((1, DI, D), lambda j, m: (m[j], 0, 0)),
                pl.BlockSpec((1, D, DI), lambda j, m: (m[j], 0, 0)),
            ],
            out_specs=pl.BlockSpec((M, D), lambda j, m: (j, 0)),
            scratch_shapes=[
                pltpu.VMEM((DI, D), jnp.bfloat16),
                pltpu.VMEM((DI, D), jnp.bfloat16),
                pltpu.VMEM((D, DI), jnp.bfloat16),
            ],
        ),
        out_shape=jax.ShapeDtypeStruct((NT, D), jnp.float32),
    )(meta, xs2, w_gate, w_up, w_down)


def _combine_body(y_hbm, pos_hbm, w1_hbm, w2_hbm, out_hbm,
                  idx0_v, idx1_v, w1_v, w2_v, buf0_v, buf1_v, out_v):
    c = lax.axis_index("c")
    s = lax.axis_index("s")
    wid = c * NSUB + s
    tbase = wid * (T // (NCORE * NSUB))

    for k in range(4):
        off = tbase + 16 * k
        pltpu.sync_copy(pos_hbm.at[pl.ds(off, 16)], idx0_v)
        pltpu.sync_copy(pos_hbm.at[pl.ds(T + off, 16)], idx1_v)
        pltpu.sync_copy(w1_hbm.at[pl.ds(off, 16)], w1_v)
        pltpu.sync_copy(w2_hbm.at[pl.ds(off, 16)], w2_v)
        pltpu.sync_copy(y_hbm.at[idx0_v], buf0_v)
        pltpu.sync_copy(y_hbm.at[idx1_v], buf1_v)

        w1c = w1_v[...]
        w2c = w2_v[...]
        for t in range(16):
            w1s = w1c[t]
            w2s = w2c[t]

            @pl.loop(0, D // 16)
            def _vv(v, t=t, w1s=w1s, w2s=w2s):
                sl = pl.ds(16 * v, 16)
                out_v[t, sl] = buf0_v[t, sl] * w1s + buf1_v[t, sl] * w2s

        pltpu.sync_copy(out_v, out_hbm.at[pl.ds(off, 16)])


def _combine(y, pos, w1, w2):
    mesh = plsc.VectorSubcoreMesh(core_axis_name="c", subcore_axis_name="s")
    f = pl.kernel(
        _combine_body,
        out_type=jax.ShapeDtypeStruct((T, D), jnp.float32),
        mesh=mesh,
        scratch_types=[
            pltpu.VMEM((16,), jnp.int32),
            pltpu.VMEM((16,), jnp.int32),
            pltpu.VMEM((16,), jnp.float32),
            pltpu.VMEM((16,), jnp.float32),
            pltpu.VMEM((16, D), jnp.float32),
            pltpu.VMEM((16, D), jnp.float32),
            pltpu.VMEM((16, D), jnp.float32),
        ],
        compiler_params=_sc_params(tc_tiling=True),
    )
    return f(y, pos, w1, w2)


def kernel(hidden_states, rms_weight, router_w, w_gate, w_up, w_down):
    shape = hidden_states.shape
    x = hidden_states.reshape(T, D).astype(jnp.float32)

    xnb, logits, pe3, w13, w23 = _router(x, rms_weight.reshape(1, D), router_w)
    pe = pe3.reshape(T)
    w1 = w13.reshape(T)
    w2 = w23.reshape(T)
    xn_i32 = jax.lax.bitcast_convert_type(
        xnb.reshape(T, D // 2, 2), jnp.int32)

    xs, meta, pos = _dispatch(pe, xn_i32)
    xs_bf = jax.lax.bitcast_convert_type(xs, jnp.bfloat16).reshape(NT, D)
    y = _gmm(meta, xs_bf, w_gate, w_up, w_down)
    out = _combine(y, pos, w1, w2)
    return out.reshape(shape), logits


# speedup vs baseline: 4.3791x; 4.3791x over previous
import jax
import jax.numpy as jnp
from jax.experimental import pallas as pl
from jax.experimental.pallas import tpu as pltpu

E = 8
TOPK = 2
D = 1024
DI = 512
EPS = 1e-06

EPG = 2
CHUNK = 512


def _tdot(a, b):
    return jax.lax.dot_general(
        a, b, (((1,), (1,)), ((), ())), preferred_element_type=jnp.float32)


def _moe_body(x_ref, rmsw_ref, rw_ref, wg_ref, wu_ref, wd_ref,
              out_ref, logits_ref,
              xn_ref, cw_ref, wgb_ref, wub_ref, wdb_ref):
    step = pl.program_id(0)
    T = x_ref.shape[0]

    @pl.when(step == 0)
    def _router():
        x = x_ref[...]
        var = jnp.mean(x * x, axis=1, keepdims=True)
        xn = x * jax.lax.rsqrt(var + EPS) * rmsw_ref[...]
        logits = jax.lax.dot_general(
            xn, rw_ref[...], (((1,), (1,)), ((), ())),
            preferred_element_type=jnp.float32)
        logits_ref[...] = logits

        m = jnp.max(logits, axis=1, keepdims=True)
        ex = jnp.exp(logits - m)
        w = ex / jnp.sum(ex, axis=1, keepdims=True)
        iota = jax.lax.broadcasted_iota(jnp.int32, (T, E), 1)
        m1 = jnp.max(w, axis=1, keepdims=True)
        i1 = jnp.min(jnp.where(w == m1, iota, E), axis=1, keepdims=True)
        oh1 = iota == i1
        w2 = jnp.where(oh1, -jnp.inf, w)
        m2 = jnp.max(w2, axis=1, keepdims=True)
        i2 = jnp.min(jnp.where(w2 == m2, iota, E), axis=1, keepdims=True)
        cw_ref[...] = jnp.where(oh1 | (iota == i2), w, 0.0)

        xn_ref[...] = xn.astype(jnp.bfloat16)
        out_ref[...] = jnp.zeros((T, D), jnp.float32)

    for k in range(EPG):
        wgb_ref[k] = wg_ref[k].astype(jnp.bfloat16)
        wub_ref[k] = wu_ref[k].astype(jnp.bfloat16)

    eio = jax.lax.broadcasted_iota(jnp.int32, (E, EPG), 0)
    oh = jnp.where(eio == step * EPG + jax.lax.broadcasted_iota(
        jnp.int32, (E, EPG), 1), 1.0, 0.0)

    for c in range(T // CHUNK):
        sl = pl.ds(c * CHUNK, CHUNK)
        xb = xn_ref[sl, :]
        wcols = jax.lax.dot_general(
            cw_ref[sl, :], oh, (((1,), (0,)), ((), ())),
            preferred_element_type=jnp.float32)
        acc = None
        for k in range(EPG):
            g = _tdot(xb, wgb_ref[k])
            u = _tdot(xb, wub_ref[k])
            if c == 0 and k == 0:
                for k2 in range(EPG):
                    wdb_ref[k2] = wd_ref[k2].astype(jnp.bfloat16)
            h = ((g * jax.nn.sigmoid(g)) * u
                 * wcols[:, k:k + 1]).astype(jnp.bfloat16)
            d = _tdot(h, wdb_ref[k])
            acc = d if acc is None else acc + d
        out_ref[sl, :] += acc


def kernel(hidden_states, rms_weight, router_w, w_gate, w_up, w_down):
    shape = hidden_states.shape
    T = shape[0] * shape[1]
    x = hidden_states.reshape(T, D).astype(jnp.float32)

    out, logits = pl.pallas_call(
        _moe_body,
        grid=(E // EPG,),
        in_specs=[
            pl.BlockSpec((T, D), lambda s: (0, 0)),
            pl.BlockSpec((1, D), lambda s: (0, 0)),
            pl.BlockSpec((E, D), lambda s: (0, 0)),
            pl.BlockSpec((EPG, DI, D), lambda s: (s, 0, 0)),
            pl.BlockSpec((EPG, DI, D), lambda s: (s, 0, 0)),
            pl.BlockSpec((EPG, D, DI), lambda s: (s, 0, 0)),
        ],
        out_specs=[
            pl.BlockSpec((T, D), lambda s: (0, 0)),
            pl.BlockSpec((T, E), lambda s: (0, 0)),
        ],
        out_shape=[
            jax.ShapeDtypeStruct((T, D), jnp.float32),
            jax.ShapeDtypeStruct((T, E), jnp.float32),
        ],
        scratch_shapes=[
            pltpu.VMEM((T, D), jnp.bfloat16),
            pltpu.VMEM((T, E), jnp.float32),
            pltpu.VMEM((EPG, DI, D), jnp.bfloat16),
            pltpu.VMEM((EPG, DI, D), jnp.bfloat16),
            pltpu.VMEM((EPG, D, DI), jnp.bfloat16),
        ],
    )(x, rms_weight.reshape(1, D), router_w, w_gate, w_up, w_down)
    return out.reshape(shape), logits
